# baseline (speedup 1.0000x reference)
import functools

import jax
import jax.numpy as jnp
from jax import lax
from jax.experimental import pallas as pl
from jax.experimental.pallas import tpu as pltpu
from jax.experimental.pallas import tpu_sc as plsc

NC = 2
NS = 16
CH = 128


def _round_up(v, m):
    return (v + m - 1) // m * m


def _sc_mesh():
    return plsc.VectorSubcoreMesh(
        core_axis_name="c", subcore_axis_name="s",
        num_cores=NC, num_subcores=NS)


def _make_sc_degree(NP, EPAD):
    nchunks = EPAD // (NC * NS * CH)
    rpt = NP // NS

    @functools.partial(
        pl.kernel,
        mesh=_sc_mesh(),
        compiler_params=pltpu.CompilerParams(use_tc_tiling_on_sc=False),
        out_type=jax.ShapeDtypeStruct((NC, NP, 8), jnp.float32),
        scratch_types=[
            pltpu.VMEM_SHARED((NP, 8), jnp.float32),
            pltpu.VMEM((CH,), jnp.int32),
            pltpu.VMEM((CH, 8), jnp.float32),
            pltpu.SemaphoreType.DMA,
        ],
    )
    def body(dst_hbm, ones_hbm, zrow_hbm, out_hbm, acc, dsti, rows, s0):
        cid = lax.axis_index("c")
        sid = lax.axis_index("s")
        w = cid * NS + sid
        def zloop(i, c):
            pltpu.sync_copy(zrow_hbm, acc.at[pl.ds(sid * rpt + i * CH, CH)])
            return c
        lax.fori_loop(0, rpt // CH, zloop, 0)
        pltpu.sync_copy(ones_hbm, rows)
        plsc.subcore_barrier()
        base = w * nchunks
        def eloop(jj, c):
            pltpu.sync_copy(dst_hbm.at[base + jj], dsti)
            pltpu.async_copy(rows, acc.at[dsti], s0, add=True).wait()
            return c
        lax.fori_loop(0, nchunks, eloop, 0)
        plsc.subcore_barrier()
        pltpu.sync_copy(acc.at[pl.ds(sid * rpt, rpt)],
                        out_hbm.at[cid, pl.ds(sid * rpt, rpt)])

    return body


def _make_sc_agg(NP, EPAD, D):
    nchunks = EPAD // (NC * NS * CH)
    rpt = NP // NS

    @functools.partial(
        pl.kernel,
        mesh=_sc_mesh(),
        compiler_params=pltpu.CompilerParams(use_tc_tiling_on_sc=False),
        out_type=jax.ShapeDtypeStruct((NC, NP, D), jnp.float32),
        scratch_types=[
            pltpu.VMEM_SHARED((NP, D), jnp.float32),
            pltpu.VMEM((CH,), jnp.int32),
            pltpu.VMEM((CH,), jnp.int32),
            pltpu.VMEM((CH,), jnp.int32),
            pltpu.VMEM((CH,), jnp.int32),
            pltpu.VMEM((CH, D), jnp.float32),
            pltpu.VMEM((CH, D), jnp.float32),
            pltpu.SemaphoreType.DMA,
            pltpu.SemaphoreType.DMA,
            pltpu.SemaphoreType.DMA,
            pltpu.SemaphoreType.DMA,
        ],
    )
    def body(z_hbm, src_hbm, dst_hbm, zrow_hbm, out_hbm,
             acc, srci0, srci1, dsti0, dsti1, rows0, rows1, g0, g1, s0, s1):
        cid = lax.axis_index("c")
        sid = lax.axis_index("s")
        w = cid * NS + sid
        def zloop(i, c):
            pltpu.sync_copy(zrow_hbm, acc.at[pl.ds(sid * rpt + i * CH, CH)])
            return c
        lax.fori_loop(0, rpt // CH, zloop, 0)
        plsc.subcore_barrier()

        base = w * nchunks
        bufs = ((srci0, dsti0, rows0, g0, s0), (srci1, dsti1, rows1, g1, s1))
        for b, (srci, dsti, rows, g, _) in enumerate(bufs):
            pltpu.sync_copy(src_hbm.at[base + b], srci)
            pltpu.async_copy(z_hbm.at[srci], rows, g)

        last = EPAD // CH - 1
        def eloop(i, c):
            for b, (srci, dsti, rows, g, s) in enumerate(bufs):
                jj = 2 * i + b
                pltpu.sync_copy(dst_hbm.at[base + jj], dsti)
                pltpu.make_async_copy(z_hbm.at[srci], rows, g).wait()
                pltpu.async_copy(rows, acc.at[dsti], s, add=True)
                nxt = jnp.minimum(base + jj + 2, last)
                pltpu.sync_copy(src_hbm.at[nxt], srci)
                pltpu.make_async_copy(rows, acc.at[dsti], s).wait()
                @pl.when(jj + 2 < nchunks)
                def _():
                    pltpu.async_copy(z_hbm.at[srci], rows, g)
            return c
        lax.fori_loop(0, nchunks // 2, eloop, 0)
        plsc.subcore_barrier()
        pltpu.sync_copy(acc.at[pl.ds(sid * rpt, rpt)],
                        out_hbm.at[cid, pl.ds(sid * rpt, rpt)])

    return body


def _dinv_from(dp_ref):
    deg = 1.0 + dp_ref[0, :, 0:1] + dp_ref[1, :, 0:1]
    return lax.rsqrt(deg)


def _tc1(xp, W1, degp, NP, BM=256):
    D_IN, H1 = W1.shape

    def body(x_ref, w_ref, dp_ref, z_ref):
        dinv = _dinv_from(dp_ref)
        z_ref[...] = jnp.dot(x_ref[...], w_ref[...],
                             preferred_element_type=jnp.float32) * dinv

    return pl.pallas_call(
        body,
        grid=(NP // BM,),
        in_specs=[
            pl.BlockSpec((BM, D_IN), lambda i: (i, 0)),
            pl.BlockSpec((D_IN, H1), lambda i: (0, 0)),
            pl.BlockSpec((2, BM, 8), lambda i: (0, i, 0)),
        ],
        out_specs=pl.BlockSpec((BM, H1), lambda i: (i, 0)),
        out_shape=jax.ShapeDtypeStruct((NP, H1), jnp.float32),
    )(xp, W1, degp)


def _tc2(z1, p1, degp, b1, W2, NP, BM=256):
    H1, H2 = W2.shape

    def body(z_ref, p_ref, dp_ref, b_ref, w_ref, o_ref):
        dinv = _dinv_from(dp_ref)
        agg = z_ref[...] + p_ref[0] + p_ref[1]
        h = jnp.maximum(agg * dinv + b_ref[...], 0.0)
        o_ref[...] = jnp.dot(h, w_ref[...],
                             preferred_element_type=jnp.float32) * dinv

    return pl.pallas_call(
        body,
        grid=(NP // BM,),
        in_specs=[
            pl.BlockSpec((BM, H1), lambda i: (i, 0)),
            pl.BlockSpec((2, BM, H1), lambda i: (0, i, 0)),
            pl.BlockSpec((2, BM, 8), lambda i: (0, i, 0)),
            pl.BlockSpec((1, H1), lambda i: (0, 0)),
            pl.BlockSpec((H1, H2), lambda i: (0, 0)),
        ],
        out_specs=pl.BlockSpec((BM, H2), lambda i: (i, 0)),
        out_shape=jax.ShapeDtypeStruct((NP, H2), jnp.float32),
    )(z1, p1, degp, b1, W2)


def _tc3(z2, p2, degp, b2, Wl, bl, NP, BM=256):
    H2, D_OUT = Wl.shape

    def body(z_ref, p_ref, dp_ref, b_ref, w_ref, bl_ref, lg_ref, pr_ref):
        dinv = _dinv_from(dp_ref)
        agg = z_ref[...] + p_ref[0] + p_ref[1]
        h = jnp.maximum(agg * dinv + b_ref[...], 0.0)
        logits = jnp.dot(h, w_ref[...],
                         preferred_element_type=jnp.float32) + bl_ref[...]
        m = jnp.max(logits, axis=1, keepdims=True)
        e = jnp.exp(logits - m)
        lg_ref[...] = logits
        pr_ref[...] = e / jnp.sum(e, axis=1, keepdims=True)

    return pl.pallas_call(
        body,
        grid=(NP // BM,),
        in_specs=[
            pl.BlockSpec((BM, H2), lambda i: (i, 0)),
            pl.BlockSpec((2, BM, H2), lambda i: (0, i, 0)),
            pl.BlockSpec((2, BM, 8), lambda i: (0, i, 0)),
            pl.BlockSpec((1, H2), lambda i: (0, 0)),
            pl.BlockSpec((H2, D_OUT), lambda i: (0, 0)),
            pl.BlockSpec((1, D_OUT), lambda i: (0, 0)),
        ],
        out_specs=[
            pl.BlockSpec((BM, D_OUT), lambda i: (i, 0)),
            pl.BlockSpec((BM, D_OUT), lambda i: (i, 0)),
        ],
        out_shape=[
            jax.ShapeDtypeStruct((NP, D_OUT), jnp.float32),
            jax.ShapeDtypeStruct((NP, D_OUT), jnp.float32),
        ],
    )(z2, p2, degp, b2, Wl, bl)


def kernel(x, edge_index, W1, b1, W2, b2, Wl, bl):
    N, D_IN = x.shape
    H1 = W1.shape[1]
    H2 = W2.shape[1]
    E = edge_index.shape[1]

    NP = _round_up(N + 1, NS * CH)
    EPAD = _round_up(E, NC * NS * CH * 2)

    pad = jnp.full((EPAD - E,), N, dtype=edge_index.dtype)
    src = jnp.concatenate([edge_index[0], pad]).reshape(EPAD // CH, CH)
    dst = jnp.concatenate([edge_index[1], pad]).reshape(EPAD // CH, CH)
    xp = jnp.pad(x, ((0, NP - N), (0, 0)))

    ones8 = jnp.ones((CH, 8), jnp.float32)
    zrow8 = jnp.zeros((CH, 8), jnp.float32)
    zrow1 = jnp.zeros((CH, H1), jnp.float32)
    zrow2 = jnp.zeros((CH, H2), jnp.float32)

    degp = _make_sc_degree(NP, EPAD)(dst, ones8, zrow8)
    z1 = _tc1(xp, W1, degp, NP)
    p1 = _make_sc_agg(NP, EPAD, H1)(z1, src, dst, zrow1)
    z2 = _tc2(z1, p1, degp, b1.reshape(1, H1), W2, NP)
    p2 = _make_sc_agg(NP, EPAD, H2)(z2, src, dst, zrow2)
    logits, probs = _tc3(z2, p2, degp, b2.reshape(1, H2),
                         Wl, bl.reshape(1, -1), NP)
    return logits[:N], probs[:N]


# speedup vs baseline: 2.0882x; 2.0882x over previous
import functools

import jax
import jax.numpy as jnp
from jax import lax
from jax.experimental import pallas as pl
from jax.experimental.pallas import tpu as pltpu
from jax.experimental.pallas import tpu_sc as plsc

NC = 2
NS = 16
CH = 128


def _round_up(v, m):
    return (v + m - 1) // m * m


def _sc_mesh():
    return plsc.VectorSubcoreMesh(
        core_axis_name="c", subcore_axis_name="s",
        num_cores=NC, num_subcores=NS)


def _make_sc_degree(NP, EPAD):
    nchunks = EPAD // (NC * NS * CH)
    rpt = NP // NS

    @functools.partial(
        pl.kernel,
        mesh=_sc_mesh(),
        compiler_params=pltpu.CompilerParams(use_tc_tiling_on_sc=False),
        out_type=jax.ShapeDtypeStruct((NC, NP, 8), jnp.float32),
        scratch_types=[
            pltpu.VMEM_SHARED((NP, 8), jnp.float32),
            pltpu.VMEM((CH,), jnp.int32),
            pltpu.VMEM((CH, 8), jnp.float32),
            pltpu.SemaphoreType.DMA,
        ],
    )
    def body(dst_hbm, ones_hbm, zrow_hbm, out_hbm, acc, dsti, rows, s0):
        cid = lax.axis_index("c")
        sid = lax.axis_index("s")
        w = cid * NS + sid
        def zloop(i, c):
            pltpu.sync_copy(zrow_hbm, acc.at[pl.ds(sid * rpt + i * CH, CH)])
            return c
        lax.fori_loop(0, rpt // CH, zloop, 0)
        pltpu.sync_copy(ones_hbm, rows)
        plsc.subcore_barrier()
        base = w * nchunks
        def eloop(jj, c):
            pltpu.sync_copy(dst_hbm.at[base + jj], dsti)
            pltpu.async_copy(rows, acc.at[dsti], s0, add=True).wait()
            return c
        lax.fori_loop(0, nchunks, eloop, 0)
        plsc.subcore_barrier()
        pltpu.sync_copy(acc.at[pl.ds(sid * rpt, rpt)],
                        out_hbm.at[cid, pl.ds(sid * rpt, rpt)])

    return body


def _make_sc_agg(NP, EPAD, D):
    nchunks = EPAD // (NC * NS * CH)
    rpt = NP // NS

    @functools.partial(
        pl.kernel,
        mesh=_sc_mesh(),
        compiler_params=pltpu.CompilerParams(use_tc_tiling_on_sc=False),
        out_type=jax.ShapeDtypeStruct((NC, NP, D), jnp.float32),
        scratch_types=[
            pltpu.VMEM_SHARED((NP, D), jnp.float32),
            pltpu.VMEM((CH,), jnp.int32),
            pltpu.VMEM((CH,), jnp.int32),
            pltpu.VMEM((CH,), jnp.int32),
            pltpu.VMEM((CH,), jnp.int32),
            pltpu.VMEM((CH, D), jnp.float32),
            pltpu.VMEM((CH, D), jnp.float32),
            pltpu.SemaphoreType.DMA,
            pltpu.SemaphoreType.DMA,
            pltpu.SemaphoreType.DMA,
            pltpu.SemaphoreType.DMA,
        ],
    )
    def body(z_hbm, src_hbm, dst_hbm, zrow_hbm, out_hbm,
             acc, srci0, srci1, dsti0, dsti1, rows0, rows1, g0, g1, s0, s1):
        cid = lax.axis_index("c")
        sid = lax.axis_index("s")
        w = cid * NS + sid
        def zloop(i, c):
            pltpu.sync_copy(zrow_hbm, acc.at[pl.ds(sid * rpt + i * CH, CH)])
            return c
        lax.fori_loop(0, rpt // CH, zloop, 0)
        plsc.subcore_barrier()

        base = w * nchunks
        bufs = ((srci0, dsti0, rows0, g0, s0), (srci1, dsti1, rows1, g1, s1))
        for b, (srci, dsti, rows, g, _) in enumerate(bufs):
            pltpu.sync_copy(src_hbm.at[base + b], srci)
            pltpu.async_copy(z_hbm.at[srci], rows, g)

        last = EPAD // CH - 1
        def eloop(i, c):
            for b, (srci, dsti, rows, g, s) in enumerate(bufs):
                jj = 2 * i + b
                pltpu.sync_copy(dst_hbm.at[base + jj], dsti)
                pltpu.make_async_copy(z_hbm.at[srci], rows, g).wait()
                pltpu.async_copy(rows, acc.at[dsti], s, add=True)
                nxt = jnp.minimum(base + jj + 2, last)
                pltpu.sync_copy(src_hbm.at[nxt], srci)
                pltpu.make_async_copy(rows, acc.at[dsti], s).wait()
                @pl.when(jj + 2 < nchunks)
                def _():
                    pltpu.async_copy(z_hbm.at[srci], rows, g)
            return c
        lax.fori_loop(0, nchunks // 2, eloop, 0)
        plsc.subcore_barrier()
        pltpu.sync_copy(acc.at[pl.ds(sid * rpt, rpt)],
                        out_hbm.at[cid, pl.ds(sid * rpt, rpt)])

    return body


def _dinv_from(dp_ref):
    deg = 1.0 + dp_ref[0, :, 0:1] + dp_ref[1, :, 0:1]
    return lax.rsqrt(deg)


def _tc1(xp, W1, degp, NP, BM=256):
    D_IN, H1 = W1.shape

    def body(x_ref, w_ref, dp_ref, z_ref):
        dinv = _dinv_from(dp_ref)
        z_ref[...] = jnp.dot(x_ref[...], w_ref[...],
                             preferred_element_type=jnp.float32) * dinv

    return pl.pallas_call(
        body,
        grid=(NP // BM,),
        in_specs=[
            pl.BlockSpec((BM, D_IN), lambda i: (i, 0)),
            pl.BlockSpec((D_IN, H1), lambda i: (0, 0)),
            pl.BlockSpec((2, BM, 8), lambda i: (0, i, 0)),
        ],
        out_specs=pl.BlockSpec((BM, H1), lambda i: (i, 0)),
        out_shape=jax.ShapeDtypeStruct((NP, H1), jnp.float32),
    )(xp, W1, degp)


def _tc2(z1, p1, degp, b1, W2, NP, BM=256):
    H1, H2 = W2.shape

    def body(z_ref, p_ref, dp_ref, b_ref, w_ref, o_ref):
        dinv = _dinv_from(dp_ref)
        agg = z_ref[...] + p_ref[0] + p_ref[1]
        h = jnp.maximum(agg * dinv + b_ref[...], 0.0)
        o_ref[...] = jnp.dot(h, w_ref[...],
                             preferred_element_type=jnp.float32) * dinv

    return pl.pallas_call(
        body,
        grid=(NP // BM,),
        in_specs=[
            pl.BlockSpec((BM, H1), lambda i: (i, 0)),
            pl.BlockSpec((2, BM, H1), lambda i: (0, i, 0)),
            pl.BlockSpec((2, BM, 8), lambda i: (0, i, 0)),
            pl.BlockSpec((1, H1), lambda i: (0, 0)),
            pl.BlockSpec((H1, H2), lambda i: (0, 0)),
        ],
        out_specs=pl.BlockSpec((BM, H2), lambda i: (i, 0)),
        out_shape=jax.ShapeDtypeStruct((NP, H2), jnp.float32),
    )(z1, p1, degp, b1, W2)


def _tc3(z2, p2, degp, b2, Wl, bl, NP, BM=256):
    H2, D_OUT = Wl.shape

    def body(z_ref, p_ref, dp_ref, b_ref, w_ref, bl_ref, lg_ref, pr_ref):
        dinv = _dinv_from(dp_ref)
        agg = z_ref[...] + p_ref[0] + p_ref[1]
        h = jnp.maximum(agg * dinv + b_ref[...], 0.0)
        logits = jnp.dot(h, w_ref[...],
                         preferred_element_type=jnp.float32) + bl_ref[...]
        m = jnp.max(logits, axis=1, keepdims=True)
        e = jnp.exp(logits - m)
        lg_ref[...] = logits
        pr_ref[...] = e / jnp.sum(e, axis=1, keepdims=True)

    return pl.pallas_call(
        body,
        grid=(NP // BM,),
        in_specs=[
            pl.BlockSpec((BM, H2), lambda i: (i, 0)),
            pl.BlockSpec((2, BM, H2), lambda i: (0, i, 0)),
            pl.BlockSpec((2, BM, 8), lambda i: (0, i, 0)),
            pl.BlockSpec((1, H2), lambda i: (0, 0)),
            pl.BlockSpec((H2, D_OUT), lambda i: (0, 0)),
            pl.BlockSpec((1, D_OUT), lambda i: (0, 0)),
        ],
        out_specs=[
            pl.BlockSpec((BM, D_OUT), lambda i: (i, 0)),
            pl.BlockSpec((BM, D_OUT), lambda i: (i, 0)),
        ],
        out_shape=[
            jax.ShapeDtypeStruct((NP, D_OUT), jnp.float32),
            jax.ShapeDtypeStruct((NP, D_OUT), jnp.float32),
        ],
    )(z2, p2, degp, b2, Wl, bl)


def kernel(x, edge_index, W1, b1, W2, b2, Wl, bl):
    N, D_IN = x.shape
    H1 = W1.shape[1]
    H2 = W2.shape[1]
    E = edge_index.shape[1]

    NP = _round_up(N + 1, NS * CH)
    EPAD = _round_up(E, NC * NS * CH * 2)

    pad = (N + jnp.arange(EPAD - E, dtype=edge_index.dtype) % (NP - N))
    src = jnp.concatenate([edge_index[0], pad]).reshape(EPAD // CH, CH)
    dst = jnp.concatenate([edge_index[1], pad]).reshape(EPAD // CH, CH)
    xp = jnp.pad(x, ((0, NP - N), (0, 0)))

    ones8 = jnp.ones((CH, 8), jnp.float32)
    zrow8 = jnp.zeros((CH, 8), jnp.float32)
    zrow1 = jnp.zeros((CH, H1), jnp.float32)
    zrow2 = jnp.zeros((CH, H2), jnp.float32)

    degp = _make_sc_degree(NP, EPAD)(dst, ones8, zrow8)
    z1 = _tc1(xp, W1, degp, NP)
    p1 = _make_sc_agg(NP, EPAD, H1)(z1, src, dst, zrow1)
    z2 = _tc2(z1, p1, degp, b1.reshape(1, H1), W2, NP)
    p2 = _make_sc_agg(NP, EPAD, H2)(z2, src, dst, zrow2)
    logits, probs = _tc3(z2, p2, degp, b2.reshape(1, H2),
                         Wl, bl.reshape(1, -1), NP)
    return logits[:N], probs[:N]


# speedup vs baseline: 2.1169x; 1.0138x over previous
import functools

import jax
import jax.numpy as jnp
from jax import lax
from jax.experimental import pallas as pl
from jax.experimental.pallas import tpu as pltpu
from jax.experimental.pallas import tpu_sc as plsc

NC = 2
NS = 16
CH = 128


def _round_up(v, m):
    return (v + m - 1) // m * m


def _sc_mesh():
    return plsc.VectorSubcoreMesh(
        core_axis_name="c", subcore_axis_name="s",
        num_cores=NC, num_subcores=NS)


def _make_sc_degree(NP, EPAD):
    nchunks = EPAD // (NC * NS * CH)
    rpt = NP // NS

    @functools.partial(
        pl.kernel,
        mesh=_sc_mesh(),
        compiler_params=pltpu.CompilerParams(use_tc_tiling_on_sc=False),
        out_type=jax.ShapeDtypeStruct((NC, NP, 8), jnp.float32),
        scratch_types=[
            pltpu.VMEM_SHARED((NP, 8), jnp.float32),
            pltpu.VMEM((CH,), jnp.int32),
            pltpu.VMEM((CH,), jnp.int32),
            pltpu.VMEM((CH, 8), jnp.float32),
            pltpu.SemaphoreType.DMA,
        ],
    )
    def body(dst_hbm, ones_hbm, zrow_hbm, out_hbm, acc, dsti0, dsti1,
             rows, s0):
        cid = lax.axis_index("c")
        sid = lax.axis_index("s")
        w = cid * NS + sid
        def zloop(i, c):
            pltpu.sync_copy(zrow_hbm, acc.at[pl.ds(sid * rpt + i * CH, CH)])
            return c
        lax.fori_loop(0, rpt // CH, zloop, 0)
        pltpu.sync_copy(ones_hbm, rows)
        plsc.subcore_barrier()
        base = w * nchunks
        last = EPAD // CH - 1
        dstis = (dsti0, dsti1)
        pltpu.sync_copy(dst_hbm.at[base], dsti0)
        def eloop(i, c):
            for b in range(2):
                jj = 2 * i + b
                pltpu.async_copy(rows, acc.at[dstis[b]], s0, add=True)
                nxt = jnp.minimum(base + jj + 1, last)
                pltpu.sync_copy(dst_hbm.at[nxt], dstis[1 - b])
                pltpu.make_async_copy(rows, acc.at[dstis[b]], s0).wait()
            return c
        lax.fori_loop(0, nchunks // 2, eloop, 0)
        plsc.subcore_barrier()
        pltpu.sync_copy(acc.at[pl.ds(sid * rpt, rpt)],
                        out_hbm.at[cid, pl.ds(sid * rpt, rpt)])

    return body


def _make_sc_agg(NP, EPAD, D):
    nchunks = EPAD // (NC * NS * CH)
    rpt = NP // NS

    @functools.partial(
        pl.kernel,
        mesh=_sc_mesh(),
        compiler_params=pltpu.CompilerParams(use_tc_tiling_on_sc=False),
        out_type=jax.ShapeDtypeStruct((NC, NP, D), jnp.float32),
        scratch_types=[
            pltpu.VMEM_SHARED((NP, D), jnp.float32),
            pltpu.VMEM((CH,), jnp.int32),
            pltpu.VMEM((CH,), jnp.int32),
            pltpu.VMEM((CH,), jnp.int32),
            pltpu.VMEM((CH,), jnp.int32),
            pltpu.VMEM((CH, D), jnp.float32),
            pltpu.VMEM((CH, D), jnp.float32),
            pltpu.SemaphoreType.DMA,
            pltpu.SemaphoreType.DMA,
            pltpu.SemaphoreType.DMA,
            pltpu.SemaphoreType.DMA,
        ],
    )
    def body(z_hbm, src_hbm, dst_hbm, zrow_hbm, out_hbm,
             acc, srci0, srci1, dsti0, dsti1, rows0, rows1, g0, g1, s0, s1):
        cid = lax.axis_index("c")
        sid = lax.axis_index("s")
        w = cid * NS + sid
        def zloop(i, c):
            pltpu.sync_copy(zrow_hbm, acc.at[pl.ds(sid * rpt + i * CH, CH)])
            return c
        lax.fori_loop(0, rpt // CH, zloop, 0)
        plsc.subcore_barrier()

        base = w * nchunks
        bufs = ((srci0, dsti0, rows0, g0, s0), (srci1, dsti1, rows1, g1, s1))
        for b, (srci, dsti, rows, g, _) in enumerate(bufs):
            pltpu.sync_copy(src_hbm.at[base + b], srci)
            pltpu.async_copy(z_hbm.at[srci], rows, g)

        last = EPAD // CH - 1
        def eloop(i, c):
            for b, (srci, dsti, rows, g, s) in enumerate(bufs):
                jj = 2 * i + b
                pltpu.sync_copy(dst_hbm.at[base + jj], dsti)
                pltpu.make_async_copy(z_hbm.at[srci], rows, g).wait()
                pltpu.async_copy(rows, acc.at[dsti], s, add=True)
                nxt = jnp.minimum(base + jj + 2, last)
                pltpu.sync_copy(src_hbm.at[nxt], srci)
                pltpu.make_async_copy(rows, acc.at[dsti], s).wait()
                @pl.when(jj + 2 < nchunks)
                def _():
                    pltpu.async_copy(z_hbm.at[srci], rows, g)
            return c
        lax.fori_loop(0, nchunks // 2, eloop, 0)
        plsc.subcore_barrier()
        pltpu.sync_copy(acc.at[pl.ds(sid * rpt, rpt)],
                        out_hbm.at[cid, pl.ds(sid * rpt, rpt)])

    return body


def _dinv_from(dp_ref):
    deg = 1.0 + dp_ref[0, :, 0:1] + dp_ref[1, :, 0:1]
    return lax.rsqrt(deg)


def _tc1a(xp, W1, NP, BM=256):
    D_IN, H1 = W1.shape

    def body(x_ref, w_ref, h_ref):
        h_ref[...] = jnp.dot(x_ref[...], w_ref[...],
                             preferred_element_type=jnp.float32)

    return pl.pallas_call(
        body,
        grid=(NP // BM,),
        in_specs=[
            pl.BlockSpec((BM, D_IN), lambda i: (i, 0)),
            pl.BlockSpec((D_IN, H1), lambda i: (0, 0)),
        ],
        out_specs=pl.BlockSpec((BM, H1), lambda i: (i, 0)),
        out_shape=jax.ShapeDtypeStruct((NP, H1), jnp.float32),
    )(xp, W1)


def _tc1b(h, degp, NP, BM=256):
    H1 = h.shape[1]

    def body(h_ref, dp_ref, z_ref):
        z_ref[...] = h_ref[...] * _dinv_from(dp_ref)

    return pl.pallas_call(
        body,
        grid=(NP // BM,),
        in_specs=[
            pl.BlockSpec((BM, H1), lambda i: (i, 0)),
            pl.BlockSpec((2, BM, 8), lambda i: (0, i, 0)),
        ],
        out_specs=pl.BlockSpec((BM, H1), lambda i: (i, 0)),
        out_shape=jax.ShapeDtypeStruct((NP, H1), jnp.float32),
    )(h, degp)


def _tc2(z1, p1, degp, b1, W2, NP, BM=256):
    H1, H2 = W2.shape

    def body(z_ref, p_ref, dp_ref, b_ref, w_ref, o_ref):
        dinv = _dinv_from(dp_ref)
        agg = z_ref[...] + p_ref[0] + p_ref[1]
        h = jnp.maximum(agg * dinv + b_ref[...], 0.0)
        o_ref[...] = jnp.dot(h, w_ref[...],
                             preferred_element_type=jnp.float32) * dinv

    return pl.pallas_call(
        body,
        grid=(NP // BM,),
        in_specs=[
            pl.BlockSpec((BM, H1), lambda i: (i, 0)),
            pl.BlockSpec((2, BM, H1), lambda i: (0, i, 0)),
            pl.BlockSpec((2, BM, 8), lambda i: (0, i, 0)),
            pl.BlockSpec((1, H1), lambda i: (0, 0)),
            pl.BlockSpec((H1, H2), lambda i: (0, 0)),
        ],
        out_specs=pl.BlockSpec((BM, H2), lambda i: (i, 0)),
        out_shape=jax.ShapeDtypeStruct((NP, H2), jnp.float32),
    )(z1, p1, degp, b1, W2)


def _tc3(z2, p2, degp, b2, Wl, bl, N, BM=400):
    H2, D_OUT = Wl.shape

    def body(z_ref, p_ref, dp_ref, b_ref, w_ref, bl_ref, lg_ref, pr_ref):
        dinv = _dinv_from(dp_ref)
        agg = z_ref[...] + p_ref[0] + p_ref[1]
        h = jnp.maximum(agg * dinv + b_ref[...], 0.0)
        logits = jnp.dot(h, w_ref[...],
                         preferred_element_type=jnp.float32) + bl_ref[...]
        m = jnp.max(logits, axis=1, keepdims=True)
        e = jnp.exp(logits - m)
        lg_ref[...] = logits
        pr_ref[...] = e / jnp.sum(e, axis=1, keepdims=True)

    return pl.pallas_call(
        body,
        grid=(N // BM,),
        in_specs=[
            pl.BlockSpec((BM, H2), lambda i: (i, 0)),
            pl.BlockSpec((2, BM, H2), lambda i: (0, i, 0)),
            pl.BlockSpec((2, BM, 8), lambda i: (0, i, 0)),
            pl.BlockSpec((1, H2), lambda i: (0, 0)),
            pl.BlockSpec((H2, D_OUT), lambda i: (0, 0)),
            pl.BlockSpec((1, D_OUT), lambda i: (0, 0)),
        ],
        out_specs=[
            pl.BlockSpec((BM, D_OUT), lambda i: (i, 0)),
            pl.BlockSpec((BM, D_OUT), lambda i: (i, 0)),
        ],
        out_shape=[
            jax.ShapeDtypeStruct((N, D_OUT), jnp.float32),
            jax.ShapeDtypeStruct((N, D_OUT), jnp.float32),
        ],
    )(z2, p2, degp, b2, Wl, bl)


def kernel(x, edge_index, W1, b1, W2, b2, Wl, bl):
    N, D_IN = x.shape
    H1 = W1.shape[1]
    H2 = W2.shape[1]
    E = edge_index.shape[1]

    NP = _round_up(N + 1, NS * CH)
    EPAD = _round_up(E, NC * NS * CH * 2)

    pad = (N + jnp.arange(EPAD - E, dtype=edge_index.dtype) % (NP - N))
    src = jnp.concatenate([edge_index[0], pad]).reshape(EPAD // CH, CH)
    dst = jnp.concatenate([edge_index[1], pad]).reshape(EPAD // CH, CH)
    xp = jnp.pad(x, ((0, NP - N), (0, 0)))

    ones8 = jnp.ones((CH, 8), jnp.float32)
    zrow8 = jnp.zeros((CH, 8), jnp.float32)
    zrow1 = jnp.zeros((CH, H1), jnp.float32)
    zrow2 = jnp.zeros((CH, H2), jnp.float32)

    h = _tc1a(xp, W1, NP)
    degp = _make_sc_degree(NP, EPAD)(dst, ones8, zrow8)
    z1 = _tc1b(h, degp, NP)
    p1 = _make_sc_agg(NP, EPAD, H1)(z1, src, dst, zrow1)
    z2 = _tc2(z1, p1, degp, b1.reshape(1, H1), W2, NP)
    p2 = _make_sc_agg(NP, EPAD, H2)(z2, src, dst, zrow2)
    logits, probs = _tc3(z2, p2, degp, b2.reshape(1, H2),
                         Wl, bl.reshape(1, -1), N)
    return logits, probs


# speedup vs baseline: 2.3100x; 1.0912x over previous
import functools

import jax
import jax.numpy as jnp
from jax import lax
from jax.experimental import pallas as pl
from jax.experimental.pallas import tpu as pltpu
from jax.experimental.pallas import tpu_sc as plsc

NC = 2
NS = 16
CH = 128


def _round_up(v, m):
    return (v + m - 1) // m * m


def _sc_mesh():
    return plsc.VectorSubcoreMesh(
        core_axis_name="c", subcore_axis_name="s",
        num_cores=NC, num_subcores=NS)


def _load_idx(edges_hbm, pads_hbm, RE, which, r, buf):
    @pl.when(r < RE)
    def _():
        pltpu.sync_copy(edges_hbm.at[which, r], buf)

    @pl.when(r >= RE)
    def _():
        pltpu.sync_copy(pads_hbm.at[which, r - RE], buf)


def _make_sc_degree(NP, EPAD, RE):
    nchunks = EPAD // (NC * NS * CH)
    rpt = NP // NS

    @functools.partial(
        pl.kernel,
        mesh=_sc_mesh(),
        compiler_params=pltpu.CompilerParams(use_tc_tiling_on_sc=False),
        out_type=jax.ShapeDtypeStruct((NC, NP, 8), jnp.float32),
        scratch_types=[
            pltpu.VMEM_SHARED((NP, 8), jnp.float32),
            pltpu.VMEM((CH,), jnp.int32),
            pltpu.VMEM((CH,), jnp.int32),
            pltpu.VMEM((CH, 8), jnp.float32),
            pltpu.SemaphoreType.DMA,
        ],
    )
    def body(edges_hbm, pads_hbm, ones_hbm, zrow_hbm, out_hbm,
             acc, dsti0, dsti1, rows, s0):
        cid = lax.axis_index("c")
        sid = lax.axis_index("s")
        w = cid * NS + sid
        def zloop(i, c):
            pltpu.sync_copy(zrow_hbm, acc.at[pl.ds(sid * rpt + i * CH, CH)])
            return c
        lax.fori_loop(0, rpt // CH, zloop, 0)
        pltpu.sync_copy(ones_hbm, rows)
        plsc.subcore_barrier()
        base = w * nchunks
        last = EPAD // CH - 1
        dstis = (dsti0, dsti1)
        _load_idx(edges_hbm, pads_hbm, RE, 1, base, dsti0)
        def eloop(i, c):
            for b in range(2):
                jj = 2 * i + b
                pltpu.async_copy(rows, acc.at[dstis[b]], s0, add=True)
                nxt = jnp.minimum(base + jj + 1, last)
                _load_idx(edges_hbm, pads_hbm, RE, 1, nxt, dstis[1 - b])
                pltpu.make_async_copy(rows, acc.at[dstis[b]], s0).wait()
            return c
        lax.fori_loop(0, nchunks // 2, eloop, 0)
        plsc.subcore_barrier()
        pltpu.sync_copy(acc.at[pl.ds(sid * rpt, rpt)],
                        out_hbm.at[cid, pl.ds(sid * rpt, rpt)])

    return body


def _make_sc_agg(NP, EPAD, D, RE):
    nchunks = EPAD // (NC * NS * CH)
    rpt = NP // NS

    @functools.partial(
        pl.kernel,
        mesh=_sc_mesh(),
        compiler_params=pltpu.CompilerParams(use_tc_tiling_on_sc=False),
        out_type=jax.ShapeDtypeStruct((NC, NP, D), jnp.float32),
        scratch_types=[
            pltpu.VMEM_SHARED((NP, D), jnp.float32),
            pltpu.VMEM((CH,), jnp.int32),
            pltpu.VMEM((CH,), jnp.int32),
            pltpu.VMEM((CH,), jnp.int32),
            pltpu.VMEM((CH,), jnp.int32),
            pltpu.VMEM((CH, D), jnp.float32),
            pltpu.VMEM((CH, D), jnp.float32),
            pltpu.SemaphoreType.DMA,
            pltpu.SemaphoreType.DMA,
            pltpu.SemaphoreType.DMA,
            pltpu.SemaphoreType.DMA,
        ],
    )
    def body(z_hbm, edges_hbm, pads_hbm, zrow_hbm, out_hbm,
             acc, srci0, srci1, dsti0, dsti1, rows0, rows1, g0, g1, s0, s1):
        cid = lax.axis_index("c")
        sid = lax.axis_index("s")
        w = cid * NS + sid
        def zloop(i, c):
            pltpu.sync_copy(zrow_hbm, acc.at[pl.ds(sid * rpt + i * CH, CH)])
            return c
        lax.fori_loop(0, rpt // CH, zloop, 0)
        plsc.subcore_barrier()

        base = w * nchunks
        bufs = ((srci0, dsti0, rows0, g0, s0), (srci1, dsti1, rows1, g1, s1))
        for b, (srci, dsti, rows, g, _) in enumerate(bufs):
            _load_idx(edges_hbm, pads_hbm, RE, 0, base + b, srci)
            pltpu.async_copy(z_hbm.at[srci], rows, g)

        last = EPAD // CH - 1
        def eloop(i, c):
            for b, (srci, dsti, rows, g, s) in enumerate(bufs):
                jj = 2 * i + b
                _load_idx(edges_hbm, pads_hbm, RE, 1, base + jj, dsti)
                pltpu.make_async_copy(z_hbm.at[srci], rows, g).wait()
                pltpu.async_copy(rows, acc.at[dsti], s, add=True)
                nxt = jnp.minimum(base + jj + 2, last)
                _load_idx(edges_hbm, pads_hbm, RE, 0, nxt, srci)
                pltpu.make_async_copy(rows, acc.at[dsti], s).wait()
                @pl.when(jj + 2 < nchunks)
                def _():
                    pltpu.async_copy(z_hbm.at[srci], rows, g)
            return c
        lax.fori_loop(0, nchunks // 2, eloop, 0)
        plsc.subcore_barrier()
        pltpu.sync_copy(acc.at[pl.ds(sid * rpt, rpt)],
                        out_hbm.at[cid, pl.ds(sid * rpt, rpt)])

    return body


def _dinv_from(dp_ref):
    deg = 1.0 + dp_ref[0, :, 0:1] + dp_ref[1, :, 0:1]
    return lax.rsqrt(deg)


def _tc1(xp, W1, degp, NP, BM=512):
    D_IN, H1 = W1.shape

    def body(x_ref, w_ref, dp_ref, z_ref):
        dinv = _dinv_from(dp_ref)
        z_ref[...] = jnp.dot(x_ref[...], w_ref[...],
                             preferred_element_type=jnp.float32) * dinv

    return pl.pallas_call(
        body,
        grid=(NP // BM,),
        in_specs=[
            pl.BlockSpec((BM, D_IN), lambda i: (i, 0)),
            pl.BlockSpec((D_IN, H1), lambda i: (0, 0)),
            pl.BlockSpec((2, BM, 8), lambda i: (0, i, 0)),
        ],
        out_specs=pl.BlockSpec((BM, H1), lambda i: (i, 0)),
        out_shape=jax.ShapeDtypeStruct((NP, H1), jnp.float32),
    )(xp, W1, degp)


def _tc2(z1, p1, degp, b1, W2, NP, BM=512):
    H1, H2 = W2.shape

    def body(z_ref, p_ref, dp_ref, b_ref, w_ref, o_ref):
        dinv = _dinv_from(dp_ref)
        agg = z_ref[...] + p_ref[0] + p_ref[1]
        h = jnp.maximum(agg * dinv + b_ref[...], 0.0)
        o_ref[...] = jnp.dot(h, w_ref[...],
                             preferred_element_type=jnp.float32) * dinv

    return pl.pallas_call(
        body,
        grid=(NP // BM,),
        in_specs=[
            pl.BlockSpec((BM, H1), lambda i: (i, 0)),
            pl.BlockSpec((2, BM, H1), lambda i: (0, i, 0)),
            pl.BlockSpec((2, BM, 8), lambda i: (0, i, 0)),
            pl.BlockSpec((1, H1), lambda i: (0, 0)),
            pl.BlockSpec((H1, H2), lambda i: (0, 0)),
        ],
        out_specs=pl.BlockSpec((BM, H2), lambda i: (i, 0)),
        out_shape=jax.ShapeDtypeStruct((NP, H2), jnp.float32),
    )(z1, p1, degp, b1, W2)


def _tc3(z2, p2, degp, b2, Wl, bl, N, BM=400):
    H2, D_OUT = Wl.shape

    def body(z_ref, p_ref, dp_ref, b_ref, w_ref, bl_ref, lg_ref, pr_ref):
        dinv = _dinv_from(dp_ref)
        agg = z_ref[...] + p_ref[0] + p_ref[1]
        h = jnp.maximum(agg * dinv + b_ref[...], 0.0)
        logits = jnp.dot(h, w_ref[...],
                         preferred_element_type=jnp.float32) + bl_ref[...]
        m = jnp.max(logits, axis=1, keepdims=True)
        e = jnp.exp(logits - m)
        lg_ref[...] = logits
        pr_ref[...] = e / jnp.sum(e, axis=1, keepdims=True)

    return pl.pallas_call(
        body,
        grid=(N // BM,),
        in_specs=[
            pl.BlockSpec((BM, H2), lambda i: (i, 0)),
            pl.BlockSpec((2, BM, H2), lambda i: (0, i, 0)),
            pl.BlockSpec((2, BM, 8), lambda i: (0, i, 0)),
            pl.BlockSpec((1, H2), lambda i: (0, 0)),
            pl.BlockSpec((H2, D_OUT), lambda i: (0, 0)),
            pl.BlockSpec((1, D_OUT), lambda i: (0, 0)),
        ],
        out_specs=[
            pl.BlockSpec((BM, D_OUT), lambda i: (i, 0)),
            pl.BlockSpec((BM, D_OUT), lambda i: (i, 0)),
        ],
        out_shape=[
            jax.ShapeDtypeStruct((N, D_OUT), jnp.float32),
            jax.ShapeDtypeStruct((N, D_OUT), jnp.float32),
        ],
    )(z2, p2, degp, b2, Wl, bl)


def kernel(x, edge_index, W1, b1, W2, b2, Wl, bl):
    N, D_IN = x.shape
    H1 = W1.shape[1]
    H2 = W2.shape[1]
    E = edge_index.shape[1]

    NP = _round_up(N + 1, NS * CH)
    EPAD = _round_up(E, NC * NS * CH * 2)

    E_main = E // CH * CH
    RE = E_main // CH
    npadc = EPAD // CH - RE
    dummy = N + jnp.arange(EPAD - E, dtype=edge_index.dtype) % (NP - N)
    if npadc > 0:
        pads = jnp.concatenate(
            [edge_index[:, E_main:],
             jnp.stack([dummy, dummy])], axis=1).reshape(2, npadc, CH)
    else:
        pads = jnp.zeros((2, 1, CH), edge_index.dtype)
    edges = edge_index[:, :E_main].reshape(2, RE, CH)
    xp = jnp.pad(x, ((0, NP - N), (0, 0)))

    ones8 = jnp.ones((CH, 8), jnp.float32)
    zrow8 = jnp.zeros((CH, 8), jnp.float32)
    zrow1 = jnp.zeros((CH, H1), jnp.float32)
    zrow2 = jnp.zeros((CH, H2), jnp.float32)

    degp = _make_sc_degree(NP, EPAD, RE)(edges, pads, ones8, zrow8)
    z1 = _tc1(xp, W1, degp, NP)
    p1 = _make_sc_agg(NP, EPAD, H1, RE)(z1, edges, pads, zrow1)
    z2 = _tc2(z1, p1, degp, b1.reshape(1, H1), W2, NP)
    p2 = _make_sc_agg(NP, EPAD, H2, RE)(z2, edges, pads, zrow2)
    logits, probs = _tc3(z2, p2, degp, b2.reshape(1, H2),
                         Wl, bl.reshape(1, -1), N)
    return logits, probs
